# baseline (speedup 1.0000x reference)
import functools

import jax
import jax.numpy as jnp
from jax import lax
from jax.experimental import pallas as pl
from jax.experimental.pallas import tpu as pltpu
from jax.experimental.pallas import tpu_sc as plsc

DIM = 256
LANES = 16
CHUNK = 128
NBUF = 3


def _perm(v, idx):
    return jnp.take_along_axis(v, idx, axis=0, mode="promise_in_bounds")


@functools.lru_cache(maxsize=None)
def _make_sc_kernel(vocab, batch):
    info = plsc.get_sparse_core_info()
    nc, ns = info.num_cores, info.num_subcores
    nw = nc * ns
    assert batch % (nw * CHUNK) == 0
    b_per_w = batch // nw
    nchunks = b_per_w // CHUNK
    nvec = DIM // LANES
    mesh = plsc.VectorSubcoreMesh(core_axis_name="c", subcore_axis_name="s")

    @functools.partial(
        pl.kernel,
        mesh=mesh,
        out_type=jax.ShapeDtypeStruct((batch,), jnp.float32),
        compiler_params=pltpu.CompilerParams(needs_layout_passes=False),
        scratch_types=[
            pltpu.VMEM((b_per_w,), jnp.int32),
            pltpu.VMEM((CHUNK, DIM), jnp.float32),
            pltpu.VMEM((CHUNK, DIM), jnp.float32),
            pltpu.VMEM((CHUNK, DIM), jnp.float32),
            pltpu.VMEM((DIM,), jnp.float32),
            pltpu.VMEM((LANES,), jnp.float32),
            pltpu.VMEM((b_per_w,), jnp.float32),
            pltpu.SemaphoreType.DMA,
            pltpu.SemaphoreType.DMA,
            pltpu.SemaphoreType.DMA,
        ],
    )
    def k(table_hbm, idx_hbm, w_hbm, b_hbm, out_hbm,
          idx_v, rows0, rows1, rows2, w_v, b_v, out_v, sem0, sem1, sem2):
        wid = lax.axis_index("s") * nc + lax.axis_index("c")
        base = wid * b_per_w

        pltpu.sync_copy(idx_hbm.at[pl.ds(base, b_per_w)], idx_v)
        pltpu.sync_copy(w_hbm, w_v)
        pltpu.sync_copy(b_hbm, b_v.at[pl.ds(0, 1)])

        row_bufs = (rows0, rows1, rows2)
        sems = (sem0, sem1, sem2)

        w_regs = [w_v[pl.ds(LANES * j, LANES)] for j in range(nvec)]
        b_scalar = b_v[...][0]
        iota = lax.iota(jnp.int32, LANES)
        hs = (8, 4, 2, 1)
        xors = {h: iota ^ h for h in hs}
        masks = {h: (iota & h) == 0 for h in hs}
        bitrev = (((iota & 1) << 3) | ((iota & 2) << 1)
                  | ((iota & 4) >> 1) | ((iota & 8) >> 3))

        def comb(x, y, h):
            t1 = jnp.where(masks[h], x, _perm(y, xors[h]))
            t2 = jnp.where(masks[h], _perm(x, xors[h]), y)
            return t1 + t2

        def start(c):
            s = c % NBUF
            return pltpu.async_copy(
                table_hbm.at[idx_v.at[pl.ds(c * CHUNK, CHUNK)]],
                row_bufs[s], sems[s])

        copies = [None] * NBUF
        for c in range(min(NBUF, nchunks)):
            copies[c % NBUF] = start(c)

        for c in range(nchunks):
            s = c % NBUF
            copies[s].wait()
            rows = row_bufs[s]

            def group(g, rows=rows, c=c):
                stack = []
                for r in range(LANES):
                    row = g * LANES + r
                    a0 = rows[row, pl.ds(0, LANES)] * w_regs[0]
                    a1 = rows[row, pl.ds(LANES, LANES)] * w_regs[1]
                    for j in range(2, nvec, 2):
                        a0 = a0 + rows[row, pl.ds(LANES * j, LANES)] * w_regs[j]
                        a1 = a1 + rows[row, pl.ds(LANES * (j + 1), LANES)] * w_regs[j + 1]
                    v, lvl = a0 + a1, 0
                    while stack and stack[-1][0] == lvl:
                        _, x = stack.pop()
                        v = comb(x, v, hs[lvl])
                        lvl += 1
                    stack.append((lvl, v))
                logits = _perm(stack[0][1], bitrev) + b_scalar
                y = 1.0 / (1.0 + jnp.exp(-logits))
                out_v[pl.ds(c * CHUNK + g * LANES, LANES)] = y

            plsc.parallel_loop(0, CHUNK // LANES)(group)
            if c + NBUF < nchunks:
                copies[s] = start(c + NBUF)

        pltpu.sync_copy(out_v, out_hbm.at[pl.ds(base, b_per_w)])

    return k


def kernel(inputs, embedding_0, W, b):
    batch = inputs.shape[0]
    vocab = embedding_0.shape[0]
    idx = inputs.reshape(batch)
    w_flat = W.reshape(DIM)
    out = _make_sc_kernel(vocab, batch)(embedding_0, idx, w_flat, b)
    return out.reshape(batch, 1)


# speedup vs baseline: 1.2534x; 1.2534x over previous
import functools

import jax
import jax.numpy as jnp
from jax import lax
from jax.experimental import pallas as pl
from jax.experimental.pallas import tpu as pltpu
from jax.experimental.pallas import tpu_sc as plsc

DIM = 256
LANES = 16
CHUNK = 128
NBUF = 3


def _perm(v, idx):
    return jnp.take_along_axis(v, idx, axis=0, mode="promise_in_bounds")


@functools.lru_cache(maxsize=None)
def _make_sc_kernel(vocab, batch):
    info = plsc.get_sparse_core_info()
    nc, ns = info.num_cores, info.num_subcores
    nw = nc * ns
    assert batch % (nw * CHUNK) == 0
    b_per_w = batch // nw
    nchunks = b_per_w // CHUNK
    nvec = DIM // LANES
    mesh = plsc.VectorSubcoreMesh(core_axis_name="c", subcore_axis_name="s")

    @functools.partial(
        pl.kernel,
        mesh=mesh,
        out_type=jax.ShapeDtypeStruct((batch,), jnp.float32),
        compiler_params=pltpu.CompilerParams(needs_layout_passes=False),
        scratch_types=[
            pltpu.VMEM((b_per_w,), jnp.int32),
            pltpu.VMEM((CHUNK, DIM), jnp.float32),
            pltpu.VMEM((CHUNK, DIM), jnp.float32),
            pltpu.VMEM((CHUNK, DIM), jnp.float32),
            pltpu.VMEM((DIM,), jnp.float32),
            pltpu.VMEM((LANES,), jnp.float32),
            pltpu.VMEM((b_per_w,), jnp.float32),
            pltpu.SemaphoreType.DMA,
            pltpu.SemaphoreType.DMA,
            pltpu.SemaphoreType.DMA,
        ],
    )
    def k(table_hbm, idx_hbm, w_hbm, b_hbm, out_hbm,
          idx_v, rows0, rows1, rows2, w_v, b_v, out_v, sem0, sem1, sem2):
        wid = lax.axis_index("s") * nc + lax.axis_index("c")
        base = wid * b_per_w

        pltpu.sync_copy(idx_hbm.at[pl.ds(base, b_per_w)], idx_v)
        pltpu.sync_copy(w_hbm, w_v)
        pltpu.sync_copy(b_hbm, b_v.at[pl.ds(0, 1)])

        row_bufs = (rows0, rows1, rows2)
        sems = (sem0, sem1, sem2)

        w_regs = [w_v[pl.ds(LANES * j, LANES)] for j in range(nvec)]
        b_scalar = b_v[...][0]
        iota = lax.iota(jnp.int32, LANES)
        hs = (8, 4, 2, 1)
        xors = {h: iota ^ h for h in hs}
        masks = {h: (iota & h) == 0 for h in hs}
        bitrev = (((iota & 1) << 3) | ((iota & 2) << 1)
                  | ((iota & 4) >> 1) | ((iota & 8) >> 3))

        def comb(x, y, h):
            t1 = jnp.where(masks[h], x, _perm(y, xors[h]))
            t2 = jnp.where(masks[h], _perm(x, xors[h]), y)
            return t1 + t2

        def start(c):
            s = c % NBUF
            return pltpu.async_copy(
                table_hbm.at[idx_v.at[pl.ds(c * CHUNK, CHUNK)]],
                row_bufs[s], sems[s])

        copies = [None] * NBUF
        for c in range(min(NBUF, nchunks)):
            copies[c % NBUF] = start(c)

        for c in range(nchunks):
            s = c % NBUF
            copies[s].wait()
            rows = row_bufs[s]

            def group(g, _, rows=rows, c=c):
                base_row = g * LANES
                vecs = [rows[base_row + r, pl.ds(0, LANES)] * w_regs[0]
                        for r in range(LANES)]
                for j in range(1, nvec):
                    for r in range(LANES):
                        vecs[r] = vecs[r] + (
                            rows[base_row + r, pl.ds(LANES * j, LANES)]
                            * w_regs[j])
                for h in hs:
                    nxt = []
                    for i in range(0, len(vecs), 2):
                        nxt.append(comb(vecs[i], vecs[i + 1], h))
                    vecs = nxt
                logits = _perm(vecs[0], bitrev) + b_scalar
                y = 1.0 / (1.0 + jnp.exp(-logits))
                out_v[pl.ds(c * CHUNK + g * LANES, LANES)] = y
                return 0

            lax.fori_loop(0, CHUNK // LANES, group, 0)
            if c + NBUF < nchunks:
                copies[s] = start(c + NBUF)

        pltpu.sync_copy(out_v, out_hbm.at[pl.ds(base, b_per_w)])

    return k


def kernel(inputs, embedding_0, W, b):
    batch = inputs.shape[0]
    vocab = embedding_0.shape[0]
    idx = inputs.reshape(batch)
    w_flat = W.reshape(DIM)
    out = _make_sc_kernel(vocab, batch)(embedding_0, idx, w_flat, b)
    return out.reshape(batch, 1)
